# baseline (speedup 1.0000x reference)
import jax
import jax.numpy as jnp
from jax import lax
from jax.experimental import pallas as pl
from jax.experimental.pallas import tpu as pltpu
from jax.experimental.pallas import tpu_sc as plsc

EMB_DIM = 32
BATCH = 16384

_NC = 2
_NS = 16
_NW = _NC * _NS
_BPW = BATCH // _NW
_CHUNK = 128


def _emb_body(loc_idx_hbm, user_idx_hbm, w_loc_hbm, w_user_hbm, out_hbm,
              idx_loc_v, idx_user_v, loc_v, user_v, sem):
    wid = lax.axis_index("s") * _NC + lax.axis_index("c")
    base = wid * _BPW

    pltpu.sync_copy(loc_idx_hbm.at[pl.ds(base, _BPW)], idx_loc_v)
    pltpu.sync_copy(user_idx_hbm.at[pl.ds(base, _BPW)], idx_user_v)

    copies = []
    for j in range(_BPW // _CHUNK):
        sl = pl.ds(j * _CHUNK, _CHUNK)
        copies.append(pltpu.async_copy(
            w_loc_hbm.at[idx_loc_v.at[sl]], loc_v.at[sl], sem))
        copies.append(pltpu.async_copy(
            w_user_hbm.at[idx_user_v.at[sl]], user_v.at[sl], sem))
    for c in copies:
        c.wait()

    pltpu.sync_copy(loc_v, out_hbm.at[pl.ds(base, _BPW), pl.ds(0, 1)])
    pltpu.sync_copy(user_v, out_hbm.at[pl.ds(base, _BPW), pl.ds(1, 1)])


@jax.jit
def _run(user_location, user_item, W_loc, W_user):
    mesh = plsc.VectorSubcoreMesh(core_axis_name="c", subcore_axis_name="s")
    w_loc3, w_user3 = lax.optimization_barrier(
        (W_loc.reshape(W_loc.shape[0], 1, EMB_DIM),
         W_user.reshape(W_user.shape[0], 1, EMB_DIM)))
    out3 = pl.kernel(
        _emb_body,
        out_type=jax.ShapeDtypeStruct((BATCH, 2, EMB_DIM), jnp.float32),
        mesh=mesh,
        scratch_types=[
            pltpu.VMEM((_BPW,), jnp.int32),
            pltpu.VMEM((_BPW,), jnp.int32),
            pltpu.VMEM((_BPW, 1, EMB_DIM), jnp.float32),
            pltpu.VMEM((_BPW, 1, EMB_DIM), jnp.float32),
            pltpu.SemaphoreType.DMA,
        ],
        compiler_params=pltpu.CompilerParams(use_tc_tiling_on_sc=False),
    )(user_location, user_item, w_loc3, w_user3)
    return out3.reshape(BATCH, 2 * EMB_DIM)


def kernel(user_location, user_item, W_loc, W_user):
    return _run(user_location.astype(jnp.int32), user_item.astype(jnp.int32),
                W_loc, W_user)


# speedup vs baseline: 1.7701x; 1.7701x over previous
import jax
import jax.numpy as jnp
from jax import lax
from jax.experimental import pallas as pl
from jax.experimental.pallas import tpu as pltpu
from jax.experimental.pallas import tpu_sc as plsc

EMB_DIM = 32
BATCH = 16384

_NC = 2
_NS = 16
_NW = _NC * _NS
_BPW = BATCH // _NW
_CHUNK = 128


def _emb_body(loc_idx_hbm, user_idx_hbm, w_loc_hbm, w_user_hbm,
              out_loc_hbm, out_user_hbm,
              idx_loc_v, idx_user_v, loc_v, user_v, sem):
    wid = lax.axis_index("s") * _NC + lax.axis_index("c")
    base = wid * _BPW

    pltpu.sync_copy(loc_idx_hbm.at[pl.ds(base, _BPW)], idx_loc_v)
    pltpu.sync_copy(user_idx_hbm.at[pl.ds(base, _BPW)], idx_user_v)

    copies = []
    for j in range(_BPW // _CHUNK):
        sl = pl.ds(j * _CHUNK, _CHUNK)
        copies.append(pltpu.async_copy(
            w_loc_hbm.at[idx_loc_v.at[sl]], loc_v.at[sl], sem))
        copies.append(pltpu.async_copy(
            w_user_hbm.at[idx_user_v.at[sl]], user_v.at[sl], sem))
    for c in copies:
        c.wait()

    pltpu.sync_copy(loc_v, out_loc_hbm.at[pl.ds(base, _BPW)])
    pltpu.sync_copy(user_v, out_user_hbm.at[pl.ds(base, _BPW)])


@jax.jit
def _run(user_location, user_item, W_loc, W_user):
    mesh = plsc.VectorSubcoreMesh(core_axis_name="c", subcore_axis_name="s")
    w_loc3 = W_loc.reshape(W_loc.shape[0], 2, EMB_DIM // 2)
    w_user3 = W_user.reshape(W_user.shape[0], 2, EMB_DIM // 2)
    out_loc, out_user = pl.kernel(
        _emb_body,
        out_type=(
            jax.ShapeDtypeStruct((BATCH, 2, EMB_DIM // 2), jnp.float32),
            jax.ShapeDtypeStruct((BATCH, 2, EMB_DIM // 2), jnp.float32),
        ),
        mesh=mesh,
        scratch_types=[
            pltpu.VMEM((_BPW,), jnp.int32),
            pltpu.VMEM((_BPW,), jnp.int32),
            pltpu.VMEM((_BPW, 2, EMB_DIM // 2), jnp.float32),
            pltpu.VMEM((_BPW, 2, EMB_DIM // 2), jnp.float32),
            pltpu.SemaphoreType.DMA,
        ],
        compiler_params=pltpu.CompilerParams(use_tc_tiling_on_sc=False),
    )(user_location, user_item, w_loc3, w_user3)
    return jnp.concatenate(
        [out_loc.reshape(BATCH, EMB_DIM), out_user.reshape(BATCH, EMB_DIM)],
        axis=1)


def kernel(user_location, user_item, W_loc, W_user):
    return _run(user_location.astype(jnp.int32), user_item.astype(jnp.int32),
                W_loc, W_user)
